# baseline (speedup 1.0000x reference)
import functools

import jax
from jax import lax
import jax.numpy as jnp
from jax.experimental import pallas as pl
from jax.experimental.pallas import tpu as pltpu
from jax.experimental.pallas import tpu_sc as plsc

N, C, H, HD, I, E = 2048, 768, 12, 64, 3072, 8
C3 = 3 * C
RB = 256
QB = 512
NI = 4
IT = I // NI
BLK = 128
CAP = 2 * N + E * BLK
NB = CAP // BLK
NEG = -1e30


def _ln(x, w, b, eps=1e-6):
    m = jnp.mean(x, axis=-1, keepdims=True)
    v = jnp.mean((x - m) ** 2, axis=-1, keepdims=True)
    return (x - m) * jax.lax.rsqrt(v + eps) * w + b


def _k1(hid_ref, w_ref, bias_ref, n1w_ref, n1b_ref, out_ref, wbf_ref):
    @pl.when(pl.program_id(0) == 0)
    def _():
        w = w_ref[...]
        qrow = jax.lax.broadcasted_iota(jnp.int32, (C3, C), 0) < C
        wbf_ref[...] = jnp.where(
            qrow, w * jnp.float32(HD ** -0.5), w).astype(jnp.bfloat16)

    xn = _ln(hid_ref[...], n1w_ref[...], n1b_ref[...])
    acc = jax.lax.dot_general(
        xn.astype(jnp.bfloat16), wbf_ref[...],
        (((1,), (1,)), ((), ())), preferred_element_type=jnp.float32)
    out_ref[...] = (acc + bias_ref[...]).astype(jnp.bfloat16)


def _k2(mb_ref, q_ref, k_ref, v_ref, out_ref, kbd_ref, vbd_ref):
    bf16 = jnp.bfloat16

    @pl.when(pl.program_id(1) == 0)
    def _():
        k2 = k_ref[...]
        v2 = v_ref[...]
        z = jnp.zeros((N, HD), bf16)
        z1 = jnp.zeros((N, 1), bf16)
        z63 = jnp.zeros((N, 63), bf16)
        on = jnp.ones((N, 1), bf16)
        kbd_ref[...] = jnp.concatenate([
            jnp.concatenate([k2[:, :HD], z], axis=1),
            jnp.concatenate([z, k2[:, HD:]], axis=1)], axis=0)
        vbd_ref[...] = jnp.concatenate([
            jnp.concatenate([v2[:, :HD], on, z63, z, z1, z63], axis=1),
            jnp.concatenate([z, z1, z63, v2[:, HD:], on, z63], axis=1)],
            axis=0)

    s = jax.lax.dot_general(
        q_ref[...], kbd_ref[...],
        (((1,), (1,)), ((), ())), preferred_element_type=jnp.float32)
    s = s + mb_ref[...]
    p = jnp.exp(jnp.clip(s, -60.0, 60.0)).astype(bf16)
    ctx = jax.lax.dot_general(
        p, vbd_ref[...],
        (((1,), (0,)), ((), ())), preferred_element_type=jnp.float32)
    out_ref[...] = jnp.concatenate(
        [ctx[:, :HD] / ctx[:, HD:HD + 1],
         ctx[:, 128:128 + HD] / ctx[:, 128 + HD:128 + HD + 1]],
        axis=1).astype(bf16)


def _k3(hid_ref, ctx_ref, wp_ref, bp_ref, g1_ref, n2w_ref, n2b_ref, wg_ref,
        h_ref, x_ref, lg_ref, wpbf_ref):
    @pl.when(pl.program_id(0) == 0)
    def _():
        wpbf_ref[...] = wp_ref[...].astype(jnp.bfloat16)

    proj = jax.lax.dot_general(
        ctx_ref[...], wpbf_ref[...],
        (((1,), (1,)), ((), ())), preferred_element_type=jnp.float32)
    h = hid_ref[...] + (proj + bp_ref[...]) * g1_ref[...]
    h_ref[...] = h
    x = _ln(h, n2w_ref[...], n2b_ref[...])
    x_ref[...] = x
    lg_ref[...] = jnp.dot(x, wg_ref[...], preferred_element_type=jnp.float32)


def _k5(lg_ref, pos0_ref, pos1_ref, w0_ref, w1_ref, beb_ref, aux_ref):
    f32 = jnp.float32
    lg = lg_ref[...]
    lane = jax.lax.broadcasted_iota(jnp.int32, (N, 128), 1)
    valid = lane < E
    lgm = jnp.where(valid, lg, NEG)
    mx = jnp.max(lgm, axis=-1, keepdims=True)
    ex = jnp.where(valid, jnp.exp(lgm - mx), 0.0)
    probs = ex / jnp.sum(ex, axis=-1, keepdims=True)
    v1 = jnp.max(probs, axis=-1, keepdims=True)
    i1 = jnp.min(jnp.where(probs == v1, lane, E), axis=-1, keepdims=True)
    oh1 = lane == i1
    p2 = jnp.where(oh1 | ~valid, NEG, probs)
    v2 = jnp.max(p2, axis=-1, keepdims=True)
    i2 = jnp.min(jnp.where(p2 == v2, lane, E), axis=-1, keepdims=True)
    oh2 = lane == i2
    sw = v1 + v2
    w0_ref[...] = v1 / sw
    w1_ref[...] = v2 / sw
    me = jnp.sum(probs, axis=0, keepdims=True) / N
    ce = jnp.sum((oh1 | oh2).astype(f32), axis=0, keepdims=True) / N
    aux_ref[...] = jnp.sum(me * ce, axis=-1, keepdims=True) * E
    oh0f = oh1.astype(f32)
    oh1f = oh2.astype(f32)
    CH = 128
    ri = jax.lax.broadcasted_iota(jnp.int32, (CH, CH), 0)
    ci = jax.lax.broadcasted_iota(jnp.int32, (CH, CH), 1)
    tri = (ci < ri).astype(f32)
    carry = jnp.zeros((1, 128), f32)
    excl = [None, None]
    for half, ohf in enumerate((oh0f, oh1f)):
        pieces = []
        for cch in range(N // CH):
            blkv = ohf[cch * CH:(cch + 1) * CH]
            pieces.append(
                jnp.dot(tri, blkv, preferred_element_type=f32) + carry)
            carry = carry + jnp.sum(blkv, axis=0, keepdims=True)
        excl[half] = jnp.concatenate(pieces, axis=0)
    counts = carry
    cp = jnp.ceil(counts / BLK) * BLK
    upper = (ri < ci).astype(f32)
    poff = jnp.dot(jnp.broadcast_to(cp, (8, 128)), upper,
                   preferred_element_type=f32)[:1]
    rank0 = jnp.sum(excl[0] * oh0f, axis=-1, keepdims=True)
    rank1 = jnp.sum(excl[1] * oh1f, axis=-1, keepdims=True)
    off0 = jnp.sum(poff * oh0f, axis=-1, keepdims=True)
    off1 = jnp.sum(poff * oh1f, axis=-1, keepdims=True)
    pos0_ref[...] = (rank0 + off0).astype(jnp.int32)
    pos1_ref[...] = (rank1 + off1).astype(jnp.int32)
    sb = poff / BLK
    sbT = jnp.transpose(jnp.broadcast_to(sb, (128, 128)))
    bthr = jnp.broadcast_to(
        jax.lax.broadcasted_iota(jnp.int32, (1, 128), 1).astype(f32),
        (128, 128))
    m = ((sbT <= bthr) & (ri < E)).astype(f32)
    beb_row = jnp.sum(m, axis=0, keepdims=True) - 1.0
    prev = jnp.concatenate([beb_row[:, :1] - 1.0, beb_row[:, :-1]], axis=1)
    first_row = (beb_row != prev).astype(f32)
    bebT = jnp.transpose(jnp.broadcast_to(beb_row, (128, 128)))
    nxt_full = jnp.where(bebT > beb_row, bebT, 999.0)
    nxt_row = jnp.min(nxt_full, axis=0, keepdims=True)
    has_row = (nxt_row < 999.0).astype(f32)
    nxt_row = jnp.where(nxt_row < 999.0, nxt_row, 0.0)
    nub = jnp.broadcast_to(jnp.sum(cp, axis=-1, keepdims=True) / BLK,
                           (1, 128))
    pad = jnp.zeros((3, 128), f32)
    beb_ref[...] = jnp.concatenate(
        [beb_row, first_row, nxt_row, has_row, nub, pad],
        axis=0).astype(jnp.int32)


def _k6(beb_ref, nub_ref, xs_ref, w1_ref, b1_ref, w2_ref, b2_ref, g2_ref,
        out_ref, w1bf_ref, w2bf_ref):
    b = pl.program_id(0)
    new_expert = (b == 0) | (beb_ref[b] != beb_ref[jnp.maximum(b - 1, 0)])

    @pl.when(new_expert)
    def _():
        w1bf_ref[...] = w1_ref[0].astype(jnp.bfloat16)
        w2bf_ref[...] = w2_ref[0].astype(jnp.bfloat16)

    @pl.when(b < nub_ref[0])
    def _():
        h1 = jax.lax.dot_general(
            xs_ref[...].astype(jnp.bfloat16), w1bf_ref[...],
            (((1,), (0,)), ((), ())), preferred_element_type=jnp.float32)
        h1 = jax.nn.gelu(h1 + b1_ref[0])
        part = jax.lax.dot_general(
            h1.astype(jnp.bfloat16), w2bf_ref[...],
            (((1,), (0,)), ((), ())), preferred_element_type=jnp.float32)
        out_ref[...] = (part + b2_ref[0]) * g2_ref[...]


TPW = N // 32


def _sca(pos0_hbm, pos1_hbm, x_hbm, xs_hbm, idx0, idx1, rows_v, sem):
    wid = lax.axis_index("s") * 2 + lax.axis_index("c")
    tb = wid * TPW
    pltpu.sync_copy(pos0_hbm.at[pl.ds(tb, TPW)], idx0)
    pltpu.sync_copy(pos1_hbm.at[pl.ds(tb, TPW)], idx1)
    pltpu.sync_copy(x_hbm.at[pl.ds(tb, TPW)], rows_v)
    pltpu.async_copy(rows_v, xs_hbm.at[idx0], sem).wait()
    pltpu.async_copy(rows_v, xs_hbm.at[idx1], sem).wait()


def _scb(eo_hbm, pos0_hbm, pos1_hbm, g0_hbm, g1_hbm,
         idx0, idx1, rows_v, sem):
    wid = lax.axis_index("s") * 2 + lax.axis_index("c")
    tb = wid * TPW
    pltpu.sync_copy(pos0_hbm.at[pl.ds(tb, TPW)], idx0)
    pltpu.sync_copy(pos1_hbm.at[pl.ds(tb, TPW)], idx1)
    pltpu.async_copy(eo_hbm.at[idx0], rows_v, sem).wait()
    pltpu.sync_copy(rows_v, g0_hbm.at[pl.ds(tb, TPW)])
    pltpu.async_copy(eo_hbm.at[idx1], rows_v, sem).wait()
    pltpu.sync_copy(rows_v, g1_hbm.at[pl.ds(tb, TPW)])


def _k7(h_ref, g0_ref, g1_ref, w0_ref, w1_ref, out_ref):
    out_ref[...] = (h_ref[...] + w0_ref[...] * g0_ref[...]
                    + w1_ref[...] * g1_ref[...])


def kernel(hidden_states, attention_mask, norm1_w, norm1_b, W_qkv, q_bias,
           v_bias, W_proj, b_proj, gamma_1, gamma_2, norm2_w, norm2_b,
           W_gate, W_fc1, b_fc1, W_fc2, b_fc2):
    f32, bf16 = jnp.float32, jnp.bfloat16
    hid = hidden_states[0]
    qkv_bias = jnp.concatenate(
        [q_bias * (HD ** -0.5), jnp.zeros_like(v_bias), v_bias])[None]
    mbias = jnp.where(attention_mask, 0.0, NEG).astype(f32)
    mbias2 = jnp.concatenate([mbias, mbias], axis=1)
    wg_pad = jnp.pad(W_gate, ((0, 0), (0, 128 - E)))

    qkv = pl.pallas_call(
        _k1,
        grid=(N // RB,),
        in_specs=[
            pl.BlockSpec((RB, C), lambda b: (b, 0)),
            pl.BlockSpec((C3, C), lambda b: (0, 0)),
            pl.BlockSpec((1, C3), lambda b: (0, 0)),
            pl.BlockSpec((1, C), lambda b: (0, 0)),
            pl.BlockSpec((1, C), lambda b: (0, 0)),
        ],
        out_specs=pl.BlockSpec((RB, C3), lambda b: (b, 0)),
        out_shape=jax.ShapeDtypeStruct((N, C3), bf16),
        scratch_shapes=[pltpu.VMEM((C3, C), bf16)],
    )(hid, W_qkv, qkv_bias, norm1_w[None], norm1_b[None])

    HP = H // 2
    ctx = pl.pallas_call(
        _k2,
        grid=(HP, N // QB),
        in_specs=[
            pl.BlockSpec((1, 2 * N), lambda h, b: (0, 0)),
            pl.BlockSpec((QB, 2 * HD), lambda h, b: (b, h)),
            pl.BlockSpec((N, 2 * HD), lambda h, b: (0, HP + h)),
            pl.BlockSpec((N, 2 * HD), lambda h, b: (0, 2 * HP + h)),
        ],
        out_specs=pl.BlockSpec((QB, 2 * HD), lambda h, b: (b, h)),
        out_shape=jax.ShapeDtypeStruct((N, C), bf16),
        scratch_shapes=[
            pltpu.VMEM((2 * N, 2 * HD), bf16),
            pltpu.VMEM((2 * N, 256), bf16),
        ],
    )(mbias2, qkv, qkv, qkv)

    h, x, lg = pl.pallas_call(
        _k3,
        grid=(N // RB,),
        in_specs=[
            pl.BlockSpec((RB, C), lambda b: (b, 0)),
            pl.BlockSpec((RB, C), lambda b: (b, 0)),
            pl.BlockSpec((C, C), lambda b: (0, 0)),
            pl.BlockSpec((1, C), lambda b: (0, 0)),
            pl.BlockSpec((1, C), lambda b: (0, 0)),
            pl.BlockSpec((1, C), lambda b: (0, 0)),
            pl.BlockSpec((1, C), lambda b: (0, 0)),
            pl.BlockSpec((C, 128), lambda b: (0, 0)),
        ],
        out_specs=[
            pl.BlockSpec((RB, C), lambda b: (b, 0)),
            pl.BlockSpec((RB, C), lambda b: (b, 0)),
            pl.BlockSpec((RB, 128), lambda b: (b, 0)),
        ],
        out_shape=[
            jax.ShapeDtypeStruct((N, C), f32),
            jax.ShapeDtypeStruct((N, C), f32),
            jax.ShapeDtypeStruct((N, 128), f32),
        ],
        scratch_shapes=[pltpu.VMEM((C, C), bf16)],
    )(hid, ctx, W_proj, b_proj[None], gamma_1[None],
      norm2_w[None], norm2_b[None], wg_pad)

    pos0, pos1, w0, w1, beb, aux = pl.pallas_call(
        _k5,
        in_specs=[pl.BlockSpec((N, 128), lambda: (0, 0))],
        out_specs=[
            pl.BlockSpec((N, 1), lambda: (0, 0)),
            pl.BlockSpec((N, 1), lambda: (0, 0)),
            pl.BlockSpec((N, 1), lambda: (0, 0)),
            pl.BlockSpec((N, 1), lambda: (0, 0)),
            pl.BlockSpec((8, 128), lambda: (0, 0)),
            pl.BlockSpec((1, 1), lambda: (0, 0)),
        ],
        out_shape=[
            jax.ShapeDtypeStruct((N, 1), jnp.int32),
            jax.ShapeDtypeStruct((N, 1), jnp.int32),
            jax.ShapeDtypeStruct((N, 1), f32),
            jax.ShapeDtypeStruct((N, 1), f32),
            jax.ShapeDtypeStruct((8, 128), jnp.int32),
            jax.ShapeDtypeStruct((1, 1), f32),
        ],
    )(lg)

    pos0f = pos0[:, 0]
    pos1f = pos1[:, 0]
    mesh = plsc.VectorSubcoreMesh(core_axis_name="c", subcore_axis_name="s")

    sca = pl.kernel(
        _sca, mesh=mesh,
        out_type=jax.ShapeDtypeStruct((CAP, C), f32),
        scratch_types=[
            pltpu.VMEM((TPW,), jnp.int32),
            pltpu.VMEM((TPW,), jnp.int32),
            pltpu.VMEM((TPW, C), f32),
            pltpu.SemaphoreType.DMA,
        ],
    )
    xs = sca(pos0f, pos1f, x)

    eo = pl.pallas_call(
        _k6,
        grid_spec=pltpu.PrefetchScalarGridSpec(
            num_scalar_prefetch=2,
            grid=(NB,),
            in_specs=[
                pl.BlockSpec((BLK, C), lambda b, beb, nub: (b, 0)),
                pl.BlockSpec((1, C, I), lambda b, beb, nub: (beb[b], 0, 0)),
                pl.BlockSpec((1, 1, I), lambda b, beb, nub: (beb[b], 0, 0)),
                pl.BlockSpec((1, I, C), lambda b, beb, nub: (beb[b], 0, 0)),
                pl.BlockSpec((1, 1, C), lambda b, beb, nub: (beb[b], 0, 0)),
                pl.BlockSpec((1, C), lambda b, beb, nub: (0, 0)),
            ],
            out_specs=pl.BlockSpec((BLK, C), lambda b, beb, nub: (b, 0)),
            scratch_shapes=[
                pltpu.VMEM((C, I), jnp.bfloat16),
                pltpu.VMEM((I, C), jnp.bfloat16),
            ],
        ),
        out_shape=jax.ShapeDtypeStruct((CAP, C), f32),
    )(beb[0, :NB], beb[4, :1], xs, W_fc1, b_fc1[:, None], W_fc2,
      b_fc2[:, None], gamma_2[None])

    scb = pl.kernel(
        _scb, mesh=mesh,
        out_type=[
            jax.ShapeDtypeStruct((N, C), f32),
            jax.ShapeDtypeStruct((N, C), f32),
        ],
        scratch_types=[
            pltpu.VMEM((TPW,), jnp.int32),
            pltpu.VMEM((TPW,), jnp.int32),
            pltpu.VMEM((TPW, C), f32),
            pltpu.SemaphoreType.DMA,
        ],
    )
    g0, g1 = scb(eo, pos0f, pos1f)

    out = pl.pallas_call(
        _k7,
        grid=(N // RB,),
        in_specs=[
            pl.BlockSpec((RB, C), lambda b: (b, 0)),
            pl.BlockSpec((RB, C), lambda b: (b, 0)),
            pl.BlockSpec((RB, C), lambda b: (b, 0)),
            pl.BlockSpec((RB, 1), lambda b: (b, 0)),
            pl.BlockSpec((RB, 1), lambda b: (b, 0)),
        ],
        out_specs=pl.BlockSpec((RB, C), lambda b: (b, 0)),
        out_shape=jax.ShapeDtypeStruct((N, C), f32),
    )(h, g0, g1, w0, w1)

    return out[None], aux[0, 0]


# speedup vs baseline: 1.0562x; 1.0562x over previous
import functools

import jax
from jax import lax
import jax.numpy as jnp
from jax.experimental import pallas as pl
from jax.experimental.pallas import tpu as pltpu
from jax.experimental.pallas import tpu_sc as plsc

N, C, H, HD, I, E = 2048, 768, 12, 64, 3072, 8
C3 = 3 * C
RB = 256
QB = 1024
NI = 4
IT = I // NI
BLK = 256
CAP = 2 * N + E * BLK
NB = CAP // BLK
NEG = -1e30


def _ln(x, w, b, eps=1e-6):
    m = jnp.mean(x, axis=-1, keepdims=True)
    v = jnp.mean((x - m) ** 2, axis=-1, keepdims=True)
    return (x - m) * jax.lax.rsqrt(v + eps) * w + b


def _k1(hid_ref, w_ref, bias_ref, n1w_ref, n1b_ref, out_ref, wbf_ref):
    @pl.when(pl.program_id(0) == 0)
    def _():
        w = w_ref[...]
        qrow = jax.lax.broadcasted_iota(jnp.int32, (C3, C), 0) < C
        wbf_ref[...] = jnp.where(
            qrow, w * jnp.float32(HD ** -0.5), w).astype(jnp.bfloat16)

    xn = _ln(hid_ref[...], n1w_ref[...], n1b_ref[...])
    acc = jax.lax.dot_general(
        xn.astype(jnp.bfloat16), wbf_ref[...],
        (((1,), (1,)), ((), ())), preferred_element_type=jnp.float32)
    out_ref[...] = (acc + bias_ref[...]).astype(jnp.bfloat16)


def _k2(mb_ref, q_ref, k_ref, v_ref, out_ref, kbd_ref, vbd_ref):
    bf16 = jnp.bfloat16

    @pl.when(pl.program_id(1) == 0)
    def _():
        k2 = k_ref[...]
        v2 = v_ref[...]
        z = jnp.zeros((N, HD), bf16)
        z1 = jnp.zeros((N, 1), bf16)
        z63 = jnp.zeros((N, 63), bf16)
        on = jnp.ones((N, 1), bf16)
        kbd_ref[...] = jnp.concatenate([
            jnp.concatenate([k2[:, :HD], z], axis=1),
            jnp.concatenate([z, k2[:, HD:]], axis=1)], axis=0)
        vbd_ref[...] = jnp.concatenate([
            jnp.concatenate([v2[:, :HD], on, z63, z, z1, z63], axis=1),
            jnp.concatenate([z, z1, z63, v2[:, HD:], on, z63], axis=1)],
            axis=0)

    s = jax.lax.dot_general(
        q_ref[...], kbd_ref[...],
        (((1,), (1,)), ((), ())), preferred_element_type=jnp.float32)
    s = s + mb_ref[...]
    p = jnp.exp(jnp.clip(s, -60.0, 60.0)).astype(bf16)
    ctx = jax.lax.dot_general(
        p, vbd_ref[...],
        (((1,), (0,)), ((), ())), preferred_element_type=jnp.float32)
    out_ref[...] = jnp.concatenate(
        [ctx[:, :HD] / ctx[:, HD:HD + 1],
         ctx[:, 128:128 + HD] / ctx[:, 128 + HD:128 + HD + 1]],
        axis=1).astype(bf16)


def _k3(hid_ref, ctx_ref, wp_ref, bp_ref, g1_ref, n2w_ref, n2b_ref, wg_ref,
        h_ref, x_ref, lg_ref, wpbf_ref):
    @pl.when(pl.program_id(0) == 0)
    def _():
        wpbf_ref[...] = wp_ref[...].astype(jnp.bfloat16)

    proj = jax.lax.dot_general(
        ctx_ref[...], wpbf_ref[...],
        (((1,), (1,)), ((), ())), preferred_element_type=jnp.float32)
    h = hid_ref[...] + (proj + bp_ref[...]) * g1_ref[...]
    h_ref[...] = h
    x = _ln(h, n2w_ref[...], n2b_ref[...])
    x_ref[...] = x
    lg_ref[...] = jnp.dot(x, wg_ref[...], preferred_element_type=jnp.float32)


def _k5(lg_ref, pos0_ref, pos1_ref, w0_ref, w1_ref, beb_ref, aux_ref):
    f32 = jnp.float32
    lg = lg_ref[...]
    lane = jax.lax.broadcasted_iota(jnp.int32, (N, 128), 1)
    valid = lane < E
    lgm = jnp.where(valid, lg, NEG)
    mx = jnp.max(lgm, axis=-1, keepdims=True)
    ex = jnp.where(valid, jnp.exp(lgm - mx), 0.0)
    probs = ex / jnp.sum(ex, axis=-1, keepdims=True)
    v1 = jnp.max(probs, axis=-1, keepdims=True)
    i1 = jnp.min(jnp.where(probs == v1, lane, E), axis=-1, keepdims=True)
    oh1 = lane == i1
    p2 = jnp.where(oh1 | ~valid, NEG, probs)
    v2 = jnp.max(p2, axis=-1, keepdims=True)
    i2 = jnp.min(jnp.where(p2 == v2, lane, E), axis=-1, keepdims=True)
    oh2 = lane == i2
    sw = v1 + v2
    w0_ref[...] = v1 / sw
    w1_ref[...] = v2 / sw
    me = jnp.sum(probs, axis=0, keepdims=True) / N
    ce = jnp.sum((oh1 | oh2).astype(f32), axis=0, keepdims=True) / N
    aux_ref[...] = jnp.sum(me * ce, axis=-1, keepdims=True) * E
    oh0f = oh1.astype(f32)
    oh1f = oh2.astype(f32)
    CH = 128
    ri = jax.lax.broadcasted_iota(jnp.int32, (CH, CH), 0)
    ci = jax.lax.broadcasted_iota(jnp.int32, (CH, CH), 1)
    tri = (ci < ri).astype(f32)
    carry = jnp.zeros((1, 128), f32)
    excl = [None, None]
    for half, ohf in enumerate((oh0f, oh1f)):
        pieces = []
        for cch in range(N // CH):
            blkv = ohf[cch * CH:(cch + 1) * CH]
            pieces.append(
                jnp.dot(tri, blkv, preferred_element_type=f32) + carry)
            carry = carry + jnp.sum(blkv, axis=0, keepdims=True)
        excl[half] = jnp.concatenate(pieces, axis=0)
    counts = carry
    cp = jnp.ceil(counts / BLK) * BLK
    upper = (ri < ci).astype(f32)
    poff = jnp.dot(jnp.broadcast_to(cp, (8, 128)), upper,
                   preferred_element_type=f32)[:1]
    rank0 = jnp.sum(excl[0] * oh0f, axis=-1, keepdims=True)
    rank1 = jnp.sum(excl[1] * oh1f, axis=-1, keepdims=True)
    off0 = jnp.sum(poff * oh0f, axis=-1, keepdims=True)
    off1 = jnp.sum(poff * oh1f, axis=-1, keepdims=True)
    pos0_ref[...] = (rank0 + off0).astype(jnp.int32)
    pos1_ref[...] = (rank1 + off1).astype(jnp.int32)
    sb = poff / BLK
    sbT = jnp.transpose(jnp.broadcast_to(sb, (128, 128)))
    bthr = jnp.broadcast_to(
        jax.lax.broadcasted_iota(jnp.int32, (1, 128), 1).astype(f32),
        (128, 128))
    m = ((sbT <= bthr) & (ri < E)).astype(f32)
    beb_row = jnp.sum(m, axis=0, keepdims=True) - 1.0
    prev = jnp.concatenate([beb_row[:, :1] - 1.0, beb_row[:, :-1]], axis=1)
    first_row = (beb_row != prev).astype(f32)
    bebT = jnp.transpose(jnp.broadcast_to(beb_row, (128, 128)))
    nxt_full = jnp.where(bebT > beb_row, bebT, 999.0)
    nxt_row = jnp.min(nxt_full, axis=0, keepdims=True)
    has_row = (nxt_row < 999.0).astype(f32)
    nxt_row = jnp.where(nxt_row < 999.0, nxt_row, 0.0)
    nub = jnp.broadcast_to(jnp.sum(cp, axis=-1, keepdims=True) / BLK,
                           (1, 128))
    pad = jnp.zeros((3, 128), f32)
    beb_ref[...] = jnp.concatenate(
        [beb_row, first_row, nxt_row, has_row, nub, pad],
        axis=0).astype(jnp.int32)


def _k6(beb_ref, nub_ref, xs_ref, w1_ref, b1_ref, w2_ref, b2_ref, g2_ref,
        out_ref, w1bf_ref, w2bf_ref):
    b = pl.program_id(0)
    new_expert = (b == 0) | (beb_ref[b] != beb_ref[jnp.maximum(b - 1, 0)])

    @pl.when(new_expert)
    def _():
        w1bf_ref[...] = w1_ref[0].astype(jnp.bfloat16)
        w2bf_ref[...] = w2_ref[0].astype(jnp.bfloat16)

    @pl.when(b < nub_ref[0])
    def _():
        h1 = jax.lax.dot_general(
            xs_ref[...].astype(jnp.bfloat16), w1bf_ref[...],
            (((1,), (0,)), ((), ())), preferred_element_type=jnp.float32)
        h1 = jax.nn.gelu(h1 + b1_ref[0])
        part = jax.lax.dot_general(
            h1.astype(jnp.bfloat16), w2bf_ref[...],
            (((1,), (0,)), ((), ())), preferred_element_type=jnp.float32)
        out_ref[...] = (part + b2_ref[0]) * g2_ref[...]


TPW = N // 32


def _sca(pos0_hbm, pos1_hbm, x_hbm, xs_hbm, idx0, idx1, rows_v, sem):
    wid = lax.axis_index("s") * 2 + lax.axis_index("c")
    tb = wid * TPW
    pltpu.sync_copy(pos0_hbm.at[pl.ds(tb, TPW)], idx0)
    pltpu.sync_copy(pos1_hbm.at[pl.ds(tb, TPW)], idx1)
    pltpu.sync_copy(x_hbm.at[pl.ds(tb, TPW)], rows_v)
    pltpu.async_copy(rows_v, xs_hbm.at[idx0], sem).wait()
    pltpu.async_copy(rows_v, xs_hbm.at[idx1], sem).wait()


def _scb(eo_hbm, pos0_hbm, pos1_hbm, g0_hbm, g1_hbm,
         idx0, idx1, rows_v, sem):
    wid = lax.axis_index("s") * 2 + lax.axis_index("c")
    tb = wid * TPW
    pltpu.sync_copy(pos0_hbm.at[pl.ds(tb, TPW)], idx0)
    pltpu.sync_copy(pos1_hbm.at[pl.ds(tb, TPW)], idx1)
    pltpu.async_copy(eo_hbm.at[idx0], rows_v, sem).wait()
    pltpu.sync_copy(rows_v, g0_hbm.at[pl.ds(tb, TPW)])
    pltpu.async_copy(eo_hbm.at[idx1], rows_v, sem).wait()
    pltpu.sync_copy(rows_v, g1_hbm.at[pl.ds(tb, TPW)])


def _k7(h_ref, g0_ref, g1_ref, w0_ref, w1_ref, out_ref):
    out_ref[...] = (h_ref[...] + w0_ref[...] * g0_ref[...]
                    + w1_ref[...] * g1_ref[...])


def kernel(hidden_states, attention_mask, norm1_w, norm1_b, W_qkv, q_bias,
           v_bias, W_proj, b_proj, gamma_1, gamma_2, norm2_w, norm2_b,
           W_gate, W_fc1, b_fc1, W_fc2, b_fc2):
    f32, bf16 = jnp.float32, jnp.bfloat16
    hid = hidden_states[0]
    qkv_bias = jnp.concatenate(
        [q_bias * (HD ** -0.5), jnp.zeros_like(v_bias), v_bias])[None]
    mbias = jnp.where(attention_mask, 0.0, NEG).astype(f32)
    mbias2 = jnp.concatenate([mbias, mbias], axis=1)
    wg_pad = jnp.pad(W_gate, ((0, 0), (0, 128 - E)))

    qkv = pl.pallas_call(
        _k1,
        grid=(N // RB,),
        in_specs=[
            pl.BlockSpec((RB, C), lambda b: (b, 0)),
            pl.BlockSpec((C3, C), lambda b: (0, 0)),
            pl.BlockSpec((1, C3), lambda b: (0, 0)),
            pl.BlockSpec((1, C), lambda b: (0, 0)),
            pl.BlockSpec((1, C), lambda b: (0, 0)),
        ],
        out_specs=pl.BlockSpec((RB, C3), lambda b: (b, 0)),
        out_shape=jax.ShapeDtypeStruct((N, C3), bf16),
        scratch_shapes=[pltpu.VMEM((C3, C), bf16)],
    )(hid, W_qkv, qkv_bias, norm1_w[None], norm1_b[None])

    HP = H // 2
    ctx = pl.pallas_call(
        _k2,
        grid=(HP, N // QB),
        in_specs=[
            pl.BlockSpec((1, 2 * N), lambda h, b: (0, 0)),
            pl.BlockSpec((QB, 2 * HD), lambda h, b: (b, h)),
            pl.BlockSpec((N, 2 * HD), lambda h, b: (0, HP + h)),
            pl.BlockSpec((N, 2 * HD), lambda h, b: (0, 2 * HP + h)),
        ],
        out_specs=pl.BlockSpec((QB, 2 * HD), lambda h, b: (b, h)),
        out_shape=jax.ShapeDtypeStruct((N, C), bf16),
        scratch_shapes=[
            pltpu.VMEM((2 * N, 2 * HD), bf16),
            pltpu.VMEM((2 * N, 256), bf16),
        ],
    )(mbias2, qkv, qkv, qkv)

    h, x, lg = pl.pallas_call(
        _k3,
        grid=(N // RB,),
        in_specs=[
            pl.BlockSpec((RB, C), lambda b: (b, 0)),
            pl.BlockSpec((RB, C), lambda b: (b, 0)),
            pl.BlockSpec((C, C), lambda b: (0, 0)),
            pl.BlockSpec((1, C), lambda b: (0, 0)),
            pl.BlockSpec((1, C), lambda b: (0, 0)),
            pl.BlockSpec((1, C), lambda b: (0, 0)),
            pl.BlockSpec((1, C), lambda b: (0, 0)),
            pl.BlockSpec((C, 128), lambda b: (0, 0)),
        ],
        out_specs=[
            pl.BlockSpec((RB, C), lambda b: (b, 0)),
            pl.BlockSpec((RB, C), lambda b: (b, 0)),
            pl.BlockSpec((RB, 128), lambda b: (b, 0)),
        ],
        out_shape=[
            jax.ShapeDtypeStruct((N, C), f32),
            jax.ShapeDtypeStruct((N, C), f32),
            jax.ShapeDtypeStruct((N, 128), f32),
        ],
        scratch_shapes=[pltpu.VMEM((C, C), bf16)],
    )(hid, ctx, W_proj, b_proj[None], gamma_1[None],
      norm2_w[None], norm2_b[None], wg_pad)

    pos0, pos1, w0, w1, beb, aux = pl.pallas_call(
        _k5,
        in_specs=[pl.BlockSpec((N, 128), lambda: (0, 0))],
        out_specs=[
            pl.BlockSpec((N, 1), lambda: (0, 0)),
            pl.BlockSpec((N, 1), lambda: (0, 0)),
            pl.BlockSpec((N, 1), lambda: (0, 0)),
            pl.BlockSpec((N, 1), lambda: (0, 0)),
            pl.BlockSpec((8, 128), lambda: (0, 0)),
            pl.BlockSpec((1, 1), lambda: (0, 0)),
        ],
        out_shape=[
            jax.ShapeDtypeStruct((N, 1), jnp.int32),
            jax.ShapeDtypeStruct((N, 1), jnp.int32),
            jax.ShapeDtypeStruct((N, 1), f32),
            jax.ShapeDtypeStruct((N, 1), f32),
            jax.ShapeDtypeStruct((8, 128), jnp.int32),
            jax.ShapeDtypeStruct((1, 1), f32),
        ],
    )(lg)

    pos0f = pos0[:, 0]
    pos1f = pos1[:, 0]
    mesh = plsc.VectorSubcoreMesh(core_axis_name="c", subcore_axis_name="s")

    sca = pl.kernel(
        _sca, mesh=mesh,
        out_type=jax.ShapeDtypeStruct((CAP, C), f32),
        scratch_types=[
            pltpu.VMEM((TPW,), jnp.int32),
            pltpu.VMEM((TPW,), jnp.int32),
            pltpu.VMEM((TPW, C), f32),
            pltpu.SemaphoreType.DMA,
        ],
    )
    xs = sca(pos0f, pos1f, x)

    eo = pl.pallas_call(
        _k6,
        grid_spec=pltpu.PrefetchScalarGridSpec(
            num_scalar_prefetch=2,
            grid=(NB,),
            in_specs=[
                pl.BlockSpec((BLK, C), lambda b, beb, nub: (b, 0)),
                pl.BlockSpec((1, C, I), lambda b, beb, nub: (beb[b], 0, 0)),
                pl.BlockSpec((1, 1, I), lambda b, beb, nub: (beb[b], 0, 0)),
                pl.BlockSpec((1, I, C), lambda b, beb, nub: (beb[b], 0, 0)),
                pl.BlockSpec((1, 1, C), lambda b, beb, nub: (beb[b], 0, 0)),
                pl.BlockSpec((1, C), lambda b, beb, nub: (0, 0)),
            ],
            out_specs=pl.BlockSpec((BLK, C), lambda b, beb, nub: (b, 0)),
            scratch_shapes=[
                pltpu.VMEM((C, I), jnp.bfloat16),
                pltpu.VMEM((I, C), jnp.bfloat16),
            ],
        ),
        out_shape=jax.ShapeDtypeStruct((CAP, C), f32),
    )(beb[0, :NB], beb[4, :1], xs, W_fc1, b_fc1[:, None], W_fc2,
      b_fc2[:, None], gamma_2[None])

    scb = pl.kernel(
        _scb, mesh=mesh,
        out_type=[
            jax.ShapeDtypeStruct((N, C), f32),
            jax.ShapeDtypeStruct((N, C), f32),
        ],
        scratch_types=[
            pltpu.VMEM((TPW,), jnp.int32),
            pltpu.VMEM((TPW,), jnp.int32),
            pltpu.VMEM((TPW, C), f32),
            pltpu.SemaphoreType.DMA,
        ],
    )
    g0, g1 = scb(eo, pos0f, pos1f)

    out = pl.pallas_call(
        _k7,
        grid=(N // RB,),
        in_specs=[
            pl.BlockSpec((RB, C), lambda b: (b, 0)),
            pl.BlockSpec((RB, C), lambda b: (b, 0)),
            pl.BlockSpec((RB, C), lambda b: (b, 0)),
            pl.BlockSpec((RB, 1), lambda b: (b, 0)),
            pl.BlockSpec((RB, 1), lambda b: (b, 0)),
        ],
        out_specs=pl.BlockSpec((RB, C), lambda b: (b, 0)),
        out_shape=jax.ShapeDtypeStruct((N, C), f32),
    )(h, g0, g1, w0, w1)

    return out[None], aux[0, 0]


# speedup vs baseline: 1.0616x; 1.0052x over previous
import functools

import jax
from jax import lax
import jax.numpy as jnp
from jax.experimental import pallas as pl
from jax.experimental.pallas import tpu as pltpu
from jax.experimental.pallas import tpu_sc as plsc

N, C, H, HD, I, E = 2048, 768, 12, 64, 3072, 8
C3 = 3 * C
RB = 256
QB = 1024
NI = 4
IT = I // NI
BLK = 256
CAP = 2 * N + E * BLK
NB = CAP // BLK
NEG = -1e30


def _ln(x, w, b, eps=1e-6):
    m = jnp.mean(x, axis=-1, keepdims=True)
    v = jnp.mean((x - m) ** 2, axis=-1, keepdims=True)
    return (x - m) * jax.lax.rsqrt(v + eps) * w + b


def _k1(hid_ref, w_ref, bias_ref, n1w_ref, n1b_ref, out_ref, wbf_ref):
    @pl.when(pl.program_id(0) == 0)
    def _():
        w = w_ref[...]
        qrow = jax.lax.broadcasted_iota(jnp.int32, (C3, C), 0) < C
        wbf_ref[...] = jnp.where(
            qrow, w * jnp.float32(HD ** -0.5), w).astype(jnp.bfloat16)

    xn = _ln(hid_ref[...], n1w_ref[...], n1b_ref[...])
    acc = jax.lax.dot_general(
        xn.astype(jnp.bfloat16), wbf_ref[...],
        (((1,), (1,)), ((), ())), preferred_element_type=jnp.float32)
    out_ref[...] = (acc + bias_ref[...]).astype(jnp.bfloat16)


def _k2(mb_ref, q_ref, k_ref, v_ref, out_ref, kbd_ref, vbd_ref):
    bf16 = jnp.bfloat16

    @pl.when(pl.program_id(1) == 0)
    def _():
        k2 = k_ref[...]
        v2 = v_ref[...]
        z = jnp.zeros((N, HD), bf16)
        z1 = jnp.zeros((N, 1), bf16)
        z63 = jnp.zeros((N, 63), bf16)
        on = jnp.ones((N, 1), bf16)
        kbd_ref[...] = jnp.concatenate([
            jnp.concatenate([k2[:, :HD], z], axis=1),
            jnp.concatenate([z, k2[:, HD:]], axis=1)], axis=0)
        vbd_ref[...] = jnp.concatenate([
            jnp.concatenate([v2[:, :HD], on, z63, z, z1, z63], axis=1),
            jnp.concatenate([z, z1, z63, v2[:, HD:], on, z63], axis=1)],
            axis=0)

    s = jax.lax.dot_general(
        q_ref[...], kbd_ref[...],
        (((1,), (1,)), ((), ())), preferred_element_type=jnp.float32)
    s = s + mb_ref[...]
    p = jnp.exp(jnp.clip(s, -60.0, 60.0)).astype(bf16)
    ctx = jax.lax.dot_general(
        p, vbd_ref[...],
        (((1,), (0,)), ((), ())), preferred_element_type=jnp.float32)
    out_ref[...] = jnp.concatenate(
        [ctx[:, :HD] / ctx[:, HD:HD + 1],
         ctx[:, 128:128 + HD] / ctx[:, 128 + HD:128 + HD + 1]],
        axis=1).astype(bf16)


def _k3(hid_ref, ctx_ref, wp_ref, bp_ref, g1_ref, n2w_ref, n2b_ref, wg_ref,
        h_ref, x_ref, pos0_ref, pos1_ref, w0_ref, w1_ref, beb_ref, aux_ref,
        wpbf_ref, lg_ref):
    b = pl.program_id(0)

    @pl.when(b == 0)
    def _():
        wpbf_ref[...] = wp_ref[...].astype(jnp.bfloat16)

    proj = jax.lax.dot_general(
        ctx_ref[...], wpbf_ref[...],
        (((1,), (1,)), ((), ())), preferred_element_type=jnp.float32)
    h = hid_ref[...] + (proj + bp_ref[...]) * g1_ref[...]
    h_ref[...] = h
    x = _ln(h, n2w_ref[...], n2b_ref[...])
    x_ref[...] = x
    lg_ref[pl.ds(b * RB, RB), :] = jnp.dot(
        x, wg_ref[...], preferred_element_type=jnp.float32)

    @pl.when(b == N // RB - 1)
    def _():
        _router(lg_ref, pos0_ref, pos1_ref, w0_ref, w1_ref, beb_ref, aux_ref)


def _router(lg_ref, pos0_ref, pos1_ref, w0_ref, w1_ref, beb_ref, aux_ref):
    f32 = jnp.float32
    lg = lg_ref[...]
    lane = jax.lax.broadcasted_iota(jnp.int32, (N, 128), 1)
    valid = lane < E
    lgm = jnp.where(valid, lg, NEG)
    mx = jnp.max(lgm, axis=-1, keepdims=True)
    ex = jnp.where(valid, jnp.exp(lgm - mx), 0.0)
    probs = ex / jnp.sum(ex, axis=-1, keepdims=True)
    v1 = jnp.max(probs, axis=-1, keepdims=True)
    i1 = jnp.min(jnp.where(probs == v1, lane, E), axis=-1, keepdims=True)
    oh1 = lane == i1
    p2 = jnp.where(oh1 | ~valid, NEG, probs)
    v2 = jnp.max(p2, axis=-1, keepdims=True)
    i2 = jnp.min(jnp.where(p2 == v2, lane, E), axis=-1, keepdims=True)
    oh2 = lane == i2
    sw = v1 + v2
    w0_ref[...] = v1 / sw
    w1_ref[...] = v2 / sw
    me = jnp.sum(probs, axis=0, keepdims=True) / N
    ce = jnp.sum((oh1 | oh2).astype(f32), axis=0, keepdims=True) / N
    aux_ref[...] = jnp.sum(me * ce, axis=-1, keepdims=True) * E
    oh0f = oh1.astype(f32)
    oh1f = oh2.astype(f32)
    CH = 128
    ri = jax.lax.broadcasted_iota(jnp.int32, (CH, CH), 0)
    ci = jax.lax.broadcasted_iota(jnp.int32, (CH, CH), 1)
    tri = (ci < ri).astype(f32)
    carry = jnp.zeros((1, 128), f32)
    excl = [None, None]
    for half, ohf in enumerate((oh0f, oh1f)):
        pieces = []
        for cch in range(N // CH):
            blkv = ohf[cch * CH:(cch + 1) * CH]
            pieces.append(
                jnp.dot(tri, blkv, preferred_element_type=f32) + carry)
            carry = carry + jnp.sum(blkv, axis=0, keepdims=True)
        excl[half] = jnp.concatenate(pieces, axis=0)
    counts = carry
    cp = jnp.ceil(counts / BLK) * BLK
    upper = (ri < ci).astype(f32)
    poff = jnp.dot(jnp.broadcast_to(cp, (8, 128)), upper,
                   preferred_element_type=f32)[:1]
    rank0 = jnp.sum(excl[0] * oh0f, axis=-1, keepdims=True)
    rank1 = jnp.sum(excl[1] * oh1f, axis=-1, keepdims=True)
    off0 = jnp.sum(poff * oh0f, axis=-1, keepdims=True)
    off1 = jnp.sum(poff * oh1f, axis=-1, keepdims=True)
    pos0_ref[...] = (rank0 + off0).astype(jnp.int32)
    pos1_ref[...] = (rank1 + off1).astype(jnp.int32)
    sb = poff / BLK
    sbT = jnp.transpose(jnp.broadcast_to(sb, (128, 128)))
    bthr = jnp.broadcast_to(
        jax.lax.broadcasted_iota(jnp.int32, (1, 128), 1).astype(f32),
        (128, 128))
    m = ((sbT <= bthr) & (ri < E)).astype(f32)
    beb_row = jnp.sum(m, axis=0, keepdims=True) - 1.0
    prev = jnp.concatenate([beb_row[:, :1] - 1.0, beb_row[:, :-1]], axis=1)
    first_row = (beb_row != prev).astype(f32)
    bebT = jnp.transpose(jnp.broadcast_to(beb_row, (128, 128)))
    nxt_full = jnp.where(bebT > beb_row, bebT, 999.0)
    nxt_row = jnp.min(nxt_full, axis=0, keepdims=True)
    has_row = (nxt_row < 999.0).astype(f32)
    nxt_row = jnp.where(nxt_row < 999.0, nxt_row, 0.0)
    nub = jnp.broadcast_to(jnp.sum(cp, axis=-1, keepdims=True) / BLK,
                           (1, 128))
    pad = jnp.zeros((3, 128), f32)
    beb_ref[...] = jnp.concatenate(
        [beb_row, first_row, nxt_row, has_row, nub, pad],
        axis=0).astype(jnp.int32)


def _k6(beb_ref, nub_ref, xs_ref, w1_ref, b1_ref, w2_ref, b2_ref, g2_ref,
        out_ref, w1bf_ref, w2bf_ref):
    b = pl.program_id(0)
    new_expert = (b == 0) | (beb_ref[b] != beb_ref[jnp.maximum(b - 1, 0)])

    @pl.when(new_expert)
    def _():
        w1bf_ref[...] = w1_ref[0].astype(jnp.bfloat16)
        w2bf_ref[...] = w2_ref[0].astype(jnp.bfloat16)

    @pl.when(b < nub_ref[0])
    def _():
        h1 = jax.lax.dot_general(
            xs_ref[...].astype(jnp.bfloat16), w1bf_ref[...],
            (((1,), (0,)), ((), ())), preferred_element_type=jnp.float32)
        h1 = jax.nn.gelu(h1 + b1_ref[0])
        part = jax.lax.dot_general(
            h1.astype(jnp.bfloat16), w2bf_ref[...],
            (((1,), (0,)), ((), ())), preferred_element_type=jnp.float32)
        out_ref[...] = (part + b2_ref[0]) * g2_ref[...]


TPW = N // 32


def _sca(pos0_hbm, pos1_hbm, x_hbm, xs_hbm, idx0, idx1, rows_v, sem):
    wid = lax.axis_index("s") * 2 + lax.axis_index("c")
    tb = wid * TPW
    pltpu.sync_copy(pos0_hbm.at[pl.ds(tb, TPW)], idx0)
    pltpu.sync_copy(pos1_hbm.at[pl.ds(tb, TPW)], idx1)
    pltpu.sync_copy(x_hbm.at[pl.ds(tb, TPW)], rows_v)
    pltpu.async_copy(rows_v, xs_hbm.at[idx0], sem).wait()
    pltpu.async_copy(rows_v, xs_hbm.at[idx1], sem).wait()


def _scb(eo_hbm, pos0_hbm, pos1_hbm, g0_hbm, g1_hbm,
         idx0, idx1, rows_v, sem):
    wid = lax.axis_index("s") * 2 + lax.axis_index("c")
    tb = wid * TPW
    pltpu.sync_copy(pos0_hbm.at[pl.ds(tb, TPW)], idx0)
    pltpu.sync_copy(pos1_hbm.at[pl.ds(tb, TPW)], idx1)
    pltpu.async_copy(eo_hbm.at[idx0], rows_v, sem).wait()
    pltpu.sync_copy(rows_v, g0_hbm.at[pl.ds(tb, TPW)])
    pltpu.async_copy(eo_hbm.at[idx1], rows_v, sem).wait()
    pltpu.sync_copy(rows_v, g1_hbm.at[pl.ds(tb, TPW)])


def _k7(h_ref, g0_ref, g1_ref, w0_ref, w1_ref, out_ref):
    out_ref[...] = (h_ref[...] + w0_ref[...] * g0_ref[...]
                    + w1_ref[...] * g1_ref[...])


def kernel(hidden_states, attention_mask, norm1_w, norm1_b, W_qkv, q_bias,
           v_bias, W_proj, b_proj, gamma_1, gamma_2, norm2_w, norm2_b,
           W_gate, W_fc1, b_fc1, W_fc2, b_fc2):
    f32, bf16 = jnp.float32, jnp.bfloat16
    hid = hidden_states[0]
    qkv_bias = jnp.concatenate(
        [q_bias * (HD ** -0.5), jnp.zeros_like(v_bias), v_bias])[None]
    mbias = jnp.where(attention_mask, 0.0, NEG).astype(f32)
    mbias2 = jnp.concatenate([mbias, mbias], axis=1)
    wg_pad = jnp.pad(W_gate, ((0, 0), (0, 128 - E)))

    qkv = pl.pallas_call(
        _k1,
        grid=(N // RB,),
        in_specs=[
            pl.BlockSpec((RB, C), lambda b: (b, 0)),
            pl.BlockSpec((C3, C), lambda b: (0, 0)),
            pl.BlockSpec((1, C3), lambda b: (0, 0)),
            pl.BlockSpec((1, C), lambda b: (0, 0)),
            pl.BlockSpec((1, C), lambda b: (0, 0)),
        ],
        out_specs=pl.BlockSpec((RB, C3), lambda b: (b, 0)),
        out_shape=jax.ShapeDtypeStruct((N, C3), bf16),
        scratch_shapes=[pltpu.VMEM((C3, C), bf16)],
    )(hid, W_qkv, qkv_bias, norm1_w[None], norm1_b[None])

    HP = H // 2
    ctx = pl.pallas_call(
        _k2,
        grid=(HP, N // QB),
        in_specs=[
            pl.BlockSpec((1, 2 * N), lambda h, b: (0, 0)),
            pl.BlockSpec((QB, 2 * HD), lambda h, b: (b, h)),
            pl.BlockSpec((N, 2 * HD), lambda h, b: (0, HP + h)),
            pl.BlockSpec((N, 2 * HD), lambda h, b: (0, 2 * HP + h)),
        ],
        out_specs=pl.BlockSpec((QB, 2 * HD), lambda h, b: (b, h)),
        out_shape=jax.ShapeDtypeStruct((N, C), bf16),
        scratch_shapes=[
            pltpu.VMEM((2 * N, 2 * HD), bf16),
            pltpu.VMEM((2 * N, 256), bf16),
        ],
    )(mbias2, qkv, qkv, qkv)

    h, x, pos0, pos1, w0, w1, beb, aux = pl.pallas_call(
        _k3,
        grid=(N // RB,),
        in_specs=[
            pl.BlockSpec((RB, C), lambda b: (b, 0)),
            pl.BlockSpec((RB, C), lambda b: (b, 0)),
            pl.BlockSpec((C, C), lambda b: (0, 0)),
            pl.BlockSpec((1, C), lambda b: (0, 0)),
            pl.BlockSpec((1, C), lambda b: (0, 0)),
            pl.BlockSpec((1, C), lambda b: (0, 0)),
            pl.BlockSpec((1, C), lambda b: (0, 0)),
            pl.BlockSpec((C, 128), lambda b: (0, 0)),
        ],
        out_specs=[
            pl.BlockSpec((RB, C), lambda b: (b, 0)),
            pl.BlockSpec((RB, C), lambda b: (b, 0)),
            pl.BlockSpec((N, 1), lambda b: (0, 0)),
            pl.BlockSpec((N, 1), lambda b: (0, 0)),
            pl.BlockSpec((N, 1), lambda b: (0, 0)),
            pl.BlockSpec((N, 1), lambda b: (0, 0)),
            pl.BlockSpec((8, 128), lambda b: (0, 0)),
            pl.BlockSpec((1, 1), lambda b: (0, 0)),
        ],
        out_shape=[
            jax.ShapeDtypeStruct((N, C), f32),
            jax.ShapeDtypeStruct((N, C), f32),
            jax.ShapeDtypeStruct((N, 1), jnp.int32),
            jax.ShapeDtypeStruct((N, 1), jnp.int32),
            jax.ShapeDtypeStruct((N, 1), f32),
            jax.ShapeDtypeStruct((N, 1), f32),
            jax.ShapeDtypeStruct((8, 128), jnp.int32),
            jax.ShapeDtypeStruct((1, 1), f32),
        ],
        scratch_shapes=[
            pltpu.VMEM((C, C), bf16),
            pltpu.VMEM((N, 128), f32),
        ],
    )(hid, ctx, W_proj, b_proj[None], gamma_1[None],
      norm2_w[None], norm2_b[None], wg_pad)

    pos0f = pos0[:, 0]
    pos1f = pos1[:, 0]
    mesh = plsc.VectorSubcoreMesh(core_axis_name="c", subcore_axis_name="s")

    sca = pl.kernel(
        _sca, mesh=mesh,
        out_type=jax.ShapeDtypeStruct((CAP, C), f32),
        scratch_types=[
            pltpu.VMEM((TPW,), jnp.int32),
            pltpu.VMEM((TPW,), jnp.int32),
            pltpu.VMEM((TPW, C), f32),
            pltpu.SemaphoreType.DMA,
        ],
    )
    xs = sca(pos0f, pos1f, x)

    eo = pl.pallas_call(
        _k6,
        grid_spec=pltpu.PrefetchScalarGridSpec(
            num_scalar_prefetch=2,
            grid=(NB,),
            in_specs=[
                pl.BlockSpec((BLK, C), lambda b, beb, nub: (b, 0)),
                pl.BlockSpec((1, C, I), lambda b, beb, nub: (beb[b], 0, 0)),
                pl.BlockSpec((1, 1, I), lambda b, beb, nub: (beb[b], 0, 0)),
                pl.BlockSpec((1, I, C), lambda b, beb, nub: (beb[b], 0, 0)),
                pl.BlockSpec((1, 1, C), lambda b, beb, nub: (beb[b], 0, 0)),
                pl.BlockSpec((1, C), lambda b, beb, nub: (0, 0)),
            ],
            out_specs=pl.BlockSpec((BLK, C), lambda b, beb, nub: (b, 0)),
            scratch_shapes=[
                pltpu.VMEM((C, I), jnp.bfloat16),
                pltpu.VMEM((I, C), jnp.bfloat16),
            ],
        ),
        out_shape=jax.ShapeDtypeStruct((CAP, C), f32),
    )(beb[0, :NB], beb[4, :1], xs, W_fc1, b_fc1[:, None], W_fc2,
      b_fc2[:, None], gamma_2[None])

    scb = pl.kernel(
        _scb, mesh=mesh,
        out_type=[
            jax.ShapeDtypeStruct((N, C), f32),
            jax.ShapeDtypeStruct((N, C), f32),
        ],
        scratch_types=[
            pltpu.VMEM((TPW,), jnp.int32),
            pltpu.VMEM((TPW,), jnp.int32),
            pltpu.VMEM((TPW, C), f32),
            pltpu.SemaphoreType.DMA,
        ],
    )
    g0, g1 = scb(eo, pos0f, pos1f)

    out = pl.pallas_call(
        _k7,
        grid=(N // RB,),
        in_specs=[
            pl.BlockSpec((RB, C), lambda b: (b, 0)),
            pl.BlockSpec((RB, C), lambda b: (b, 0)),
            pl.BlockSpec((RB, C), lambda b: (b, 0)),
            pl.BlockSpec((RB, 1), lambda b: (b, 0)),
            pl.BlockSpec((RB, 1), lambda b: (b, 0)),
        ],
        out_specs=pl.BlockSpec((RB, C), lambda b: (b, 0)),
        out_shape=jax.ShapeDtypeStruct((N, C), f32),
    )(h, g0, g1, w0, w1)

    return out[None], aux[0, 0]
